# baseline (speedup 1.0000x reference)
import functools

import jax
import jax.numpy as jnp
from jax import lax
from jax.experimental import pallas as pl
from jax.experimental.pallas import tpu as pltpu
from jax.experimental.pallas import tpu_sc as plsc

NC = 2
NS = 16
CH = 64
DW = 16


def _cdiv(a, b):
    return (a + b - 1) // b


def _make_agg(n_pad, d, pc, nph0, nph1, with_deg):
    rpt = n_pad // NS
    npairs = pc // 2

    out_type = [jax.ShapeDtypeStruct((NC * n_pad, d), jnp.float32)]
    scratch = [
        pltpu.VMEM((pc, CH), jnp.int32),
        pltpu.VMEM((pc, CH), jnp.int32),
        pltpu.VMEM((CH, d), jnp.float32),
        pltpu.VMEM((CH, d), jnp.float32),
        pltpu.VMEM_SHARED((n_pad, d), jnp.float32),
        pltpu.SemaphoreType.DMA,
        pltpu.SemaphoreType.DMA,
    ]
    if with_deg:
        out_type.append(jax.ShapeDtypeStruct((NC * n_pad, DW), jnp.float32))
        scratch += [
            pltpu.VMEM((CH, DW), jnp.float32),
            pltpu.VMEM_SHARED((n_pad, DW), jnp.float32),
        ]

    def body(f_hbm, src_hbm, dst_hbm, zrow_hbm, zdeg_hbm, ones_hbm, *rest):
        if with_deg:
            (agg_hbm, deg_hbm, src_v, dst_v, rows_a, rows_b, acc_sh,
             sem_a, sem_b, ones_v, dacc_sh) = rest
        else:
            (agg_hbm, src_v, dst_v, rows_a, rows_b, acc_sh,
             sem_a, sem_b) = rest
            deg_hbm = ones_v = dacc_sh = None
        c = lax.axis_index("c")
        s = lax.axis_index("s")
        nph_c = jnp.where(c == 0, nph0, nph1)
        base0 = jnp.where(c == 0, s * (nph0 * pc),
                          NS * (nph0 * pc) + s * (nph1 * pc))
        obase = c * n_pad + s * rpt

        pltpu.sync_copy(zrow_hbm, acc_sh.at[pl.ds(s * rpt, rpt)])
        if with_deg:
            pltpu.sync_copy(zdeg_hbm, dacc_sh.at[pl.ds(s * rpt, rpt)])
            pltpu.sync_copy(ones_hbm, ones_v)

        plsc.subcore_barrier()

        def gather_wait(buf, sem):
            pltpu.make_async_copy(f_hbm.at[pl.ds(0, CH)], buf, sem).wait()

        def process(j, buf, sem):
            gather_wait(buf, sem)
            pltpu.sync_copy(buf, acc_sh.at[dst_v.at[j]], add=True)
            if with_deg:
                pltpu.sync_copy(ones_v, dacc_sh.at[dst_v.at[j]], add=True)

        def phase(h, _):
            @pl.when(h < nph_c)
            def _():
                base = base0 + h * pc
                pltpu.sync_copy(src_hbm.at[pl.ds(base, pc)], src_v)
                pltpu.sync_copy(dst_hbm.at[pl.ds(base, pc)], dst_v)

                pltpu.async_copy(f_hbm.at[src_v.at[0]], rows_a, sem_a)
                pltpu.async_copy(f_hbm.at[src_v.at[1]], rows_b, sem_b)

                def pair(p, _):
                    j0 = 2 * p
                    process(j0, rows_a, sem_a)

                    @pl.when(p < npairs - 1)
                    def _():
                        pltpu.async_copy(f_hbm.at[src_v.at[j0 + 2]], rows_a,
                                         sem_a)

                    process(j0 + 1, rows_b, sem_b)

                    @pl.when(p < npairs - 1)
                    def _():
                        pltpu.async_copy(f_hbm.at[src_v.at[j0 + 3]], rows_b,
                                         sem_b)

                    return 0

                lax.fori_loop(0, npairs, pair, 0)

            return 0

        lax.fori_loop(0, max(nph0, nph1), phase, 0)

        plsc.subcore_barrier()

        pltpu.sync_copy(acc_sh.at[pl.ds(s * rpt, rpt)],
                        agg_hbm.at[pl.ds(obase, rpt)])
        if with_deg:
            pltpu.sync_copy(dacc_sh.at[pl.ds(s * rpt, rpt)],
                            deg_hbm.at[pl.ds(obase, rpt)])

    mesh = plsc.VectorSubcoreMesh(core_axis_name="c", subcore_axis_name="s",
                                  num_cores=NC)
    return pl.kernel(body, out_type=out_type, mesh=mesh,
                     scratch_types=scratch,
                     compiler_params=pltpu.CompilerParams(
                         use_tc_tiling_on_sc=False))


def _mm_first(xp, w, b, tm):
    n_pad, d = xp.shape

    def body(x_ref, w_ref, b_ref, o_ref):
        h = jnp.dot(x_ref[...], w_ref[...], preferred_element_type=jnp.float32)
        o_ref[...] = jnp.maximum(h + b_ref[...], 0.0)

    return pl.pallas_call(
        body,
        grid=(n_pad // tm,),
        in_specs=[
            pl.BlockSpec((tm, d), lambda i: (i, 0)),
            pl.BlockSpec((d, d), lambda i: (0, 0)),
            pl.BlockSpec((1, d), lambda i: (0, 0)),
        ],
        out_specs=pl.BlockSpec((tm, d), lambda i: (i, 0)),
        out_shape=jax.ShapeDtypeStruct((n_pad, d), jnp.float32),
    )(xp, w, b.reshape(1, d))


def _mm_pool(p, dg, w, b, tm):
    _, n_pad, d = p.shape

    def body(p_ref, dg_ref, w_ref, b_ref, o_ref):
        pv = p_ref[...]
        dv = dg_ref[...]
        deg = jnp.maximum(dv[0, :, 0:1] + dv[1, :, 0:1], 1.0)
        h = (pv[0] + pv[1]) / deg
        h = jnp.dot(h, w_ref[...], preferred_element_type=jnp.float32)
        o_ref[...] = jnp.maximum(h + b_ref[...], 0.0)

    return pl.pallas_call(
        body,
        grid=(n_pad // tm,),
        in_specs=[
            pl.BlockSpec((NC, tm, d), lambda i: (0, i, 0)),
            pl.BlockSpec((NC, tm, DW), lambda i: (0, i, 0)),
            pl.BlockSpec((d, d), lambda i: (0, 0)),
            pl.BlockSpec((1, d), lambda i: (0, 0)),
        ],
        out_specs=pl.BlockSpec((tm, d), lambda i: (i, 0)),
        out_shape=jax.ShapeDtypeStruct((n_pad, d), jnp.float32),
    )(p, dg, w, b.reshape(1, d))


def _pool_final(p, dg, tm):
    _, n_pad, d = p.shape

    def body(p_ref, dg_ref, o_ref):
        pv = p_ref[...]
        dv = dg_ref[...]
        deg = jnp.maximum(dv[0, :, 0:1] + dv[1, :, 0:1], 1.0)
        o_ref[...] = (pv[0] + pv[1]) / deg

    return pl.pallas_call(
        body,
        grid=(n_pad // tm,),
        in_specs=[
            pl.BlockSpec((NC, tm, d), lambda i: (0, i, 0)),
            pl.BlockSpec((NC, tm, DW), lambda i: (0, i, 0)),
        ],
        out_specs=pl.BlockSpec((tm, d), lambda i: (i, 0)),
        out_shape=jax.ShapeDtypeStruct((n_pad, d), jnp.float32),
    )(p, dg)


def kernel(x, edge_index, W1, b1, W2, b2, W3, b3):
    n, d = x.shape
    e = edge_index.shape[1]
    n_pad = _cdiv(n, NS * CH) * NS * CH
    pc = 40
    nphases = _cdiv(e, NS * CH * pc)
    nph0, nph1 = 5, 3
    assert nph0 + nph1 == nphases
    c_chunks = pc * nphases
    e_pad = c_chunks * NS * CH
    tm = n_pad // 8

    src = edge_index[0]
    dst = edge_index[1]
    srcp = jnp.concatenate(
        [src, jnp.zeros((e_pad - e,), jnp.int32)]).reshape(NS * c_chunks, CH)
    dstp = jnp.concatenate(
        [dst, jnp.full((e_pad - e,), n, jnp.int32)]).reshape(NS * c_chunks, CH)
    xp = jnp.pad(x, ((0, n_pad - n), (0, 0)))

    rpt = n_pad // NS
    zrow = jnp.zeros((rpt, d), jnp.float32)
    zdeg = jnp.zeros((rpt, DW), jnp.float32)
    ones = jnp.ones((CH, DW), jnp.float32)

    agg_deg = _make_agg(n_pad, d, pc, nph0, nph1, with_deg=True)
    agg = _make_agg(n_pad, d, pc, nph0, nph1, with_deg=False)

    f1 = _mm_first(xp, W1, b1, tm)
    p1, dg = agg_deg(f1, srcp, dstp, zrow, zdeg, ones)
    p1 = p1.reshape(NC, n_pad, d)
    dg = dg.reshape(NC, n_pad, DW)
    f2 = _mm_pool(p1, dg, W2, b2, tm)
    (p2,) = agg(f2, srcp, dstp, zrow, zdeg, ones)
    f3 = _mm_pool(p2.reshape(NC, n_pad, d), dg, W3, b3, tm)
    (p3,) = agg(f3, srcp, dstp, zrow, zdeg, ones)
    out = _pool_final(p3.reshape(NC, n_pad, d), dg, tm)
    return out[:n]


# speedup vs baseline: 1.0235x; 1.0235x over previous
import functools

import jax
import jax.numpy as jnp
from jax import lax
from jax.experimental import pallas as pl
from jax.experimental.pallas import tpu as pltpu
from jax.experimental.pallas import tpu_sc as plsc

NC = 2
NS = 16
CH = 64
DW = 16


def _cdiv(a, b):
    return (a + b - 1) // b


def _make_agg(n_pad, d, pc, nph0, nph1, with_deg):
    rpt = n_pad // NS
    npairs = pc // 2

    out_type = [jax.ShapeDtypeStruct((NC * n_pad, d), jnp.float32)]
    scratch = [
        pltpu.VMEM((pc, CH), jnp.int32),
        pltpu.VMEM((pc, CH), jnp.int32),
        pltpu.VMEM((CH, d), jnp.float32),
        pltpu.VMEM((CH, d), jnp.float32),
        pltpu.VMEM_SHARED((n_pad, d), jnp.float32),
        pltpu.SemaphoreType.DMA,
        pltpu.SemaphoreType.DMA,
    ]
    if with_deg:
        out_type.append(jax.ShapeDtypeStruct((NC * n_pad, DW), jnp.float32))
        scratch += [
            pltpu.VMEM((CH, DW), jnp.float32),
            pltpu.VMEM_SHARED((n_pad, DW), jnp.float32),
        ]

    def body(f_hbm, src_hbm, dst_hbm, zrow_hbm, zdeg_hbm, ones_hbm, *rest):
        if with_deg:
            (agg_hbm, deg_hbm, src_v, dst_v, rows_a, rows_b, acc_sh,
             sem_a, sem_b, ones_v, dacc_sh) = rest
        else:
            (agg_hbm, src_v, dst_v, rows_a, rows_b, acc_sh,
             sem_a, sem_b) = rest
            deg_hbm = ones_v = dacc_sh = None
        c = lax.axis_index("c")
        s = lax.axis_index("s")
        nph_c = jnp.where(c == 0, nph0, nph1)
        base0 = jnp.where(c == 0, s * (nph0 * pc),
                          NS * (nph0 * pc) + s * (nph1 * pc))
        obase = c * n_pad + s * rpt

        pltpu.sync_copy(zrow_hbm, acc_sh.at[pl.ds(s * rpt, rpt)])
        if with_deg:
            pltpu.sync_copy(zdeg_hbm, dacc_sh.at[pl.ds(s * rpt, rpt)])
            pltpu.sync_copy(ones_hbm, ones_v)

        plsc.subcore_barrier()

        def gather_wait(buf, sem):
            pltpu.make_async_copy(f_hbm.at[pl.ds(0, CH)], buf, sem).wait()

        def process(j, buf, sem):
            gather_wait(buf, sem)
            pltpu.sync_copy(buf, acc_sh.at[dst_v.at[j]], add=True)
            if with_deg:
                pltpu.sync_copy(ones_v, dacc_sh.at[dst_v.at[j]], add=True)

        def phase(h, _):
            @pl.when(h < nph_c)
            def _():
                base = base0 + h * pc
                pltpu.sync_copy(src_hbm.at[pl.ds(base, pc)], src_v)
                pltpu.sync_copy(dst_hbm.at[pl.ds(base, pc)], dst_v)

                pltpu.async_copy(f_hbm.at[src_v.at[0]], rows_a, sem_a)
                pltpu.async_copy(f_hbm.at[src_v.at[1]], rows_b, sem_b)

                def pair(p, _):
                    j0 = 2 * p
                    process(j0, rows_a, sem_a)

                    @pl.when(p < npairs - 1)
                    def _():
                        pltpu.async_copy(f_hbm.at[src_v.at[j0 + 2]], rows_a,
                                         sem_a)

                    process(j0 + 1, rows_b, sem_b)

                    @pl.when(p < npairs - 1)
                    def _():
                        pltpu.async_copy(f_hbm.at[src_v.at[j0 + 3]], rows_b,
                                         sem_b)

                    return 0

                lax.fori_loop(0, npairs, pair, 0)

            return 0

        lax.fori_loop(0, max(nph0, nph1), phase, 0)

        plsc.subcore_barrier()

        pltpu.sync_copy(acc_sh.at[pl.ds(s * rpt, rpt)],
                        agg_hbm.at[pl.ds(obase, rpt)])
        if with_deg:
            pltpu.sync_copy(dacc_sh.at[pl.ds(s * rpt, rpt)],
                            deg_hbm.at[pl.ds(obase, rpt)])

    mesh = plsc.VectorSubcoreMesh(core_axis_name="c", subcore_axis_name="s",
                                  num_cores=NC)
    return pl.kernel(body, out_type=out_type, mesh=mesh,
                     scratch_types=scratch,
                     compiler_params=pltpu.CompilerParams(
                         use_tc_tiling_on_sc=False))


def _mm_first(xp, w, b, tm):
    n_pad, d = xp.shape

    def body(x_ref, w_ref, b_ref, o_ref):
        h = jnp.dot(x_ref[...], w_ref[...], preferred_element_type=jnp.float32)
        o_ref[...] = jnp.maximum(h + b_ref[...], 0.0)

    return pl.pallas_call(
        body,
        grid=(n_pad // tm,),
        in_specs=[
            pl.BlockSpec((tm, d), lambda i: (i, 0)),
            pl.BlockSpec((d, d), lambda i: (0, 0)),
            pl.BlockSpec((1, d), lambda i: (0, 0)),
        ],
        out_specs=pl.BlockSpec((tm, d), lambda i: (i, 0)),
        out_shape=jax.ShapeDtypeStruct((n_pad, d), jnp.float32),
    )(xp, w, b.reshape(1, d))


def _mm_pool(p, dg, w, b, tm):
    _, n_pad, d = p.shape

    def body(p_ref, dg_ref, w_ref, b_ref, o_ref):
        pv = p_ref[...]
        dv = dg_ref[...]
        deg = jnp.maximum(dv[0, :, 0:1] + dv[1, :, 0:1], 1.0)
        h = (pv[0] + pv[1]) / deg
        h = jnp.dot(h, w_ref[...], preferred_element_type=jnp.float32)
        o_ref[...] = jnp.maximum(h + b_ref[...], 0.0)

    return pl.pallas_call(
        body,
        grid=(n_pad // tm,),
        in_specs=[
            pl.BlockSpec((NC, tm, d), lambda i: (0, i, 0)),
            pl.BlockSpec((NC, tm, DW), lambda i: (0, i, 0)),
            pl.BlockSpec((d, d), lambda i: (0, 0)),
            pl.BlockSpec((1, d), lambda i: (0, 0)),
        ],
        out_specs=pl.BlockSpec((tm, d), lambda i: (i, 0)),
        out_shape=jax.ShapeDtypeStruct((n_pad, d), jnp.float32),
    )(p, dg, w, b.reshape(1, d))


def _pool_final(p, dg, tm):
    _, n_pad, d = p.shape

    def body(p_ref, dg_ref, o_ref):
        pv = p_ref[...]
        dv = dg_ref[...]
        deg = jnp.maximum(dv[0, :, 0:1] + dv[1, :, 0:1], 1.0)
        o_ref[...] = (pv[0] + pv[1]) / deg

    return pl.pallas_call(
        body,
        grid=(n_pad // tm,),
        in_specs=[
            pl.BlockSpec((NC, tm, d), lambda i: (0, i, 0)),
            pl.BlockSpec((NC, tm, DW), lambda i: (0, i, 0)),
        ],
        out_specs=pl.BlockSpec((tm, d), lambda i: (i, 0)),
        out_shape=jax.ShapeDtypeStruct((n_pad, d), jnp.float32),
    )(p, dg)


def kernel(x, edge_index, W1, b1, W2, b2, W3, b3):
    n, d = x.shape
    e = edge_index.shape[1]
    n_pad = _cdiv(n, NS * CH) * NS * CH
    pc = 40
    nphases = _cdiv(e, NS * CH * pc)
    c_chunks = pc * nphases
    e_pad = c_chunks * NS * CH
    tm = n_pad // 8

    src = edge_index[0]
    dst = edge_index[1]
    srcp = jnp.concatenate(
        [src, jnp.zeros((e_pad - e,), jnp.int32)]).reshape(NS * c_chunks, CH)
    dstp = jnp.concatenate(
        [dst, jnp.full((e_pad - e,), n, jnp.int32)]).reshape(NS * c_chunks, CH)
    xp = jnp.pad(x, ((0, n_pad - n), (0, 0)))

    rpt = n_pad // NS
    zrow = jnp.zeros((rpt, d), jnp.float32)
    zdeg = jnp.zeros((rpt, DW), jnp.float32)
    ones = jnp.ones((CH, DW), jnp.float32)

    agg_deg = _make_agg(n_pad, d, 40, 6, 2, with_deg=True)
    agg = _make_agg(n_pad, d, 80, 3, 1, with_deg=False)

    f1 = _mm_first(xp, W1, b1, tm)
    p1, dg = agg_deg(f1, srcp, dstp, zrow, zdeg, ones)
    p1 = p1.reshape(NC, n_pad, d)
    dg = dg.reshape(NC, n_pad, DW)
    f2 = _mm_pool(p1, dg, W2, b2, tm)
    (p2,) = agg(f2, srcp, dstp, zrow, zdeg, ones)
    f3 = _mm_pool(p2.reshape(NC, n_pad, d), dg, W3, b3, tm)
    (p3,) = agg(f3, srcp, dstp, zrow, zdeg, ones)
    out = _pool_final(p3.reshape(NC, n_pad, d), dg, tm)
    return out[:n]


# speedup vs baseline: 1.0343x; 1.0106x over previous
import functools

import jax
import jax.numpy as jnp
from jax import lax
from jax.experimental import pallas as pl
from jax.experimental.pallas import tpu as pltpu
from jax.experimental.pallas import tpu_sc as plsc

NC = 2
NS = 16
CH = 64
DW = 16


def _cdiv(a, b):
    return (a + b - 1) // b


def _make_agg(n_pad, d, pc, nph0, nph1, with_deg):
    rpt = n_pad // NS
    npairs = pc // 2

    out_type = [jax.ShapeDtypeStruct((NC * n_pad, d), jnp.float32)]
    scratch = [
        pltpu.VMEM((pc, CH), jnp.int32),
        pltpu.VMEM((pc, CH), jnp.int32),
        pltpu.VMEM((CH, d), jnp.float32),
        pltpu.VMEM((CH, d), jnp.float32),
        pltpu.VMEM_SHARED((n_pad, d), jnp.float32),
        pltpu.SemaphoreType.DMA,
        pltpu.SemaphoreType.DMA,
    ]
    if with_deg:
        out_type.append(jax.ShapeDtypeStruct((NC * n_pad, DW), jnp.float32))
        scratch += [
            pltpu.VMEM((CH, DW), jnp.float32),
            pltpu.VMEM_SHARED((n_pad, DW), jnp.float32),
        ]

    def body(f_hbm, src_hbm, dst_hbm, zrow_hbm, zdeg_hbm, ones_hbm, *rest):
        if with_deg:
            (agg_hbm, deg_hbm, src_v, dst_v, rows_a, rows_b, acc_sh,
             sem_a, sem_b, ones_v, dacc_sh) = rest
        else:
            (agg_hbm, src_v, dst_v, rows_a, rows_b, acc_sh,
             sem_a, sem_b) = rest
            deg_hbm = ones_v = dacc_sh = None
        c = lax.axis_index("c")
        s = lax.axis_index("s")
        nph_c = jnp.where(c == 0, nph0, nph1)
        base0 = jnp.where(c == 0, s * (nph0 * pc),
                          NS * (nph0 * pc) + s * (nph1 * pc))
        obase = c * n_pad + s * rpt

        pltpu.sync_copy(zrow_hbm, acc_sh.at[pl.ds(s * rpt, rpt)])
        if with_deg:
            pltpu.sync_copy(zdeg_hbm, dacc_sh.at[pl.ds(s * rpt, rpt)])
            pltpu.sync_copy(ones_hbm, ones_v)

        plsc.subcore_barrier()

        def gather_wait(buf, sem):
            pltpu.make_async_copy(f_hbm.at[pl.ds(0, CH)], buf, sem).wait()

        def process(j, buf, sem):
            gather_wait(buf, sem)
            pltpu.sync_copy(buf, acc_sh.at[dst_v.at[j]], add=True)
            if with_deg:
                pltpu.sync_copy(ones_v, dacc_sh.at[dst_v.at[j]], add=True)

        def phase(h, _):
            @pl.when(h < nph_c)
            def _():
                base = base0 + h * pc
                pltpu.sync_copy(src_hbm.at[pl.ds(base, pc)], src_v)
                pltpu.sync_copy(dst_hbm.at[pl.ds(base, pc)], dst_v)

                pltpu.async_copy(f_hbm.at[src_v.at[0]], rows_a, sem_a)
                pltpu.async_copy(f_hbm.at[src_v.at[1]], rows_b, sem_b)

                def pair(p, _):
                    j0 = 2 * p
                    process(j0, rows_a, sem_a)

                    @pl.when(p < npairs - 1)
                    def _():
                        pltpu.async_copy(f_hbm.at[src_v.at[j0 + 2]], rows_a,
                                         sem_a)

                    process(j0 + 1, rows_b, sem_b)

                    @pl.when(p < npairs - 1)
                    def _():
                        pltpu.async_copy(f_hbm.at[src_v.at[j0 + 3]], rows_b,
                                         sem_b)

                    return 0

                lax.fori_loop(0, npairs, pair, 0)

            return 0

        lax.fori_loop(0, max(nph0, nph1), phase, 0)

        plsc.subcore_barrier()

        pltpu.sync_copy(acc_sh.at[pl.ds(s * rpt, rpt)],
                        agg_hbm.at[pl.ds(obase, rpt)])
        if with_deg:
            pltpu.sync_copy(dacc_sh.at[pl.ds(s * rpt, rpt)],
                            deg_hbm.at[pl.ds(obase, rpt)])

    mesh = plsc.VectorSubcoreMesh(core_axis_name="c", subcore_axis_name="s",
                                  num_cores=NC)
    return pl.kernel(body, out_type=out_type, mesh=mesh,
                     scratch_types=scratch,
                     compiler_params=pltpu.CompilerParams(
                         use_tc_tiling_on_sc=False))


def _mm_first(xp, w, b, tm):
    n_pad, d = xp.shape

    def body(x_ref, w_ref, b_ref, o_ref):
        h = jnp.dot(x_ref[...], w_ref[...], preferred_element_type=jnp.float32)
        o_ref[...] = jnp.maximum(h + b_ref[...], 0.0)

    return pl.pallas_call(
        body,
        grid=(n_pad // tm,),
        in_specs=[
            pl.BlockSpec((tm, d), lambda i: (i, 0)),
            pl.BlockSpec((d, d), lambda i: (0, 0)),
            pl.BlockSpec((1, d), lambda i: (0, 0)),
        ],
        out_specs=pl.BlockSpec((tm, d), lambda i: (i, 0)),
        out_shape=jax.ShapeDtypeStruct((n_pad, d), jnp.float32),
    )(xp, w, b.reshape(1, d))


def _mm_pool(p, dg, w, b, tm):
    _, n_pad, d = p.shape

    def body(p_ref, dg_ref, w_ref, b_ref, o_ref):
        pv = p_ref[...]
        dv = dg_ref[...]
        deg = jnp.maximum(dv[0, :, 0:1] + dv[1, :, 0:1], 1.0)
        h = (pv[0] + pv[1]) / deg
        h = jnp.dot(h, w_ref[...], preferred_element_type=jnp.float32)
        o_ref[...] = jnp.maximum(h + b_ref[...], 0.0)

    return pl.pallas_call(
        body,
        grid=(n_pad // tm,),
        in_specs=[
            pl.BlockSpec((NC, tm, d), lambda i: (0, i, 0)),
            pl.BlockSpec((NC, tm, DW), lambda i: (0, i, 0)),
            pl.BlockSpec((d, d), lambda i: (0, 0)),
            pl.BlockSpec((1, d), lambda i: (0, 0)),
        ],
        out_specs=pl.BlockSpec((tm, d), lambda i: (i, 0)),
        out_shape=jax.ShapeDtypeStruct((n_pad, d), jnp.float32),
    )(p, dg, w, b.reshape(1, d))


def _pool_final(p, dg, tm):
    _, n_pad, d = p.shape

    def body(p_ref, dg_ref, o_ref):
        pv = p_ref[...]
        dv = dg_ref[...]
        deg = jnp.maximum(dv[0, :, 0:1] + dv[1, :, 0:1], 1.0)
        o_ref[...] = (pv[0] + pv[1]) / deg

    return pl.pallas_call(
        body,
        grid=(n_pad // tm,),
        in_specs=[
            pl.BlockSpec((NC, tm, d), lambda i: (0, i, 0)),
            pl.BlockSpec((NC, tm, DW), lambda i: (0, i, 0)),
        ],
        out_specs=pl.BlockSpec((tm, d), lambda i: (i, 0)),
        out_shape=jax.ShapeDtypeStruct((n_pad, d), jnp.float32),
    )(p, dg)


def kernel(x, edge_index, W1, b1, W2, b2, W3, b3):
    n, d = x.shape
    e = edge_index.shape[1]
    n_pad = _cdiv(n, NS * CH) * NS * CH
    pc = 40
    nphases = _cdiv(e, NS * CH * pc)
    c_chunks = pc * nphases
    e_pad = c_chunks * NS * CH
    tm = n_pad // 8

    src = edge_index[0]
    dst = edge_index[1]
    srcp = jnp.concatenate(
        [src, jnp.zeros((e_pad - e,), jnp.int32)]).reshape(NS * c_chunks, CH)
    dstp = jnp.concatenate(
        [dst, jnp.full((e_pad - e,), n, jnp.int32)]).reshape(NS * c_chunks, CH)
    xp = jnp.pad(x, ((0, n_pad - n), (0, 0)))

    rpt = n_pad // NS
    zrow = jnp.zeros((rpt, d), jnp.float32)
    zdeg = jnp.zeros((rpt, DW), jnp.float32)
    ones = jnp.ones((CH, DW), jnp.float32)

    agg_deg = _make_agg(n_pad, d, 20, 13, 3, with_deg=True)
    agg = _make_agg(n_pad, d, 20, 13, 3, with_deg=False)

    f1 = _mm_first(xp, W1, b1, tm)
    p1, dg = agg_deg(f1, srcp, dstp, zrow, zdeg, ones)
    p1 = p1.reshape(NC, n_pad, d)
    dg = dg.reshape(NC, n_pad, DW)
    f2 = _mm_pool(p1, dg, W2, b2, tm)
    (p2,) = agg(f2, srcp, dstp, zrow, zdeg, ones)
    f3 = _mm_pool(p2.reshape(NC, n_pad, d), dg, W3, b3, tm)
    (p3,) = agg(f3, srcp, dstp, zrow, zdeg, ones)
    out = _pool_final(p3.reshape(NC, n_pad, d), dg, tm)
    return out[:n]


# speedup vs baseline: 1.1601x; 1.1217x over previous
import functools

import jax
import jax.numpy as jnp
from jax import lax
from jax.experimental import pallas as pl
from jax.experimental.pallas import tpu as pltpu
from jax.experimental.pallas import tpu_sc as plsc

NC = 2
NS = 16
CH = 64
DW = 16


def _cdiv(a, b):
    return (a + b - 1) // b


def _make_agg(n_pad, d, pc, nph0, nph1, with_deg):
    rpt = n_pad // NS
    npairs = pc // 2

    out_type = [jax.ShapeDtypeStruct((NC * n_pad, d), jnp.float32)]
    scratch = [
        pltpu.VMEM((pc, CH), jnp.int32),
        pltpu.VMEM((pc, CH), jnp.int32),
        pltpu.VMEM((CH, d), jnp.float32),
        pltpu.VMEM((CH, d), jnp.float32),
        pltpu.VMEM_SHARED((n_pad, d), jnp.float32),
        pltpu.SemaphoreType.DMA,
        pltpu.SemaphoreType.DMA,
    ]
    if with_deg:
        out_type.append(jax.ShapeDtypeStruct((NC * n_pad, DW), jnp.float32))
        scratch += [
            pltpu.VMEM((CH, DW), jnp.float32),
            pltpu.VMEM_SHARED((n_pad, DW), jnp.float32),
        ]

    def body(f_hbm, src_hbm, dst_hbm, zrow_hbm, zdeg_hbm, ones_hbm, *rest):
        if with_deg:
            (agg_hbm, deg_hbm, src_v, dst_v, rows_a, rows_b, acc_sh,
             sem_a, sem_b, ones_v, dacc_sh) = rest
        else:
            (agg_hbm, src_v, dst_v, rows_a, rows_b, acc_sh,
             sem_a, sem_b) = rest
            deg_hbm = ones_v = dacc_sh = None
        c = lax.axis_index("c")
        s = lax.axis_index("s")
        nph_c = jnp.where(c == 0, nph0, nph1)
        base0 = jnp.where(c == 0, s * (nph0 * pc),
                          NS * (nph0 * pc) + s * (nph1 * pc))
        obase = c * n_pad + s * rpt

        pltpu.sync_copy(zrow_hbm, acc_sh.at[pl.ds(s * rpt, rpt)])
        if with_deg:
            pltpu.sync_copy(zdeg_hbm, dacc_sh.at[pl.ds(s * rpt, rpt)])
            pltpu.sync_copy(ones_hbm, ones_v)

        plsc.subcore_barrier()

        def gather_wait(buf, sem):
            pltpu.make_async_copy(f_hbm.at[pl.ds(0, CH)], buf, sem).wait()

        def process(j, buf, sem):
            gather_wait(buf, sem)
            pltpu.sync_copy(buf, acc_sh.at[dst_v.at[j]], add=True)
            if with_deg:
                pltpu.sync_copy(ones_v, dacc_sh.at[dst_v.at[j]], add=True)

        def phase(h, _):
            @pl.when(h < nph_c)
            def _():
                base = base0 + h * pc
                pltpu.sync_copy(src_hbm.at[pl.ds(base, pc)], src_v)
                pltpu.sync_copy(dst_hbm.at[pl.ds(base, pc)], dst_v)

                pltpu.async_copy(f_hbm.at[src_v.at[0]], rows_a, sem_a)
                pltpu.async_copy(f_hbm.at[src_v.at[1]], rows_b, sem_b)

                def pair(p, _):
                    j0 = 2 * p
                    process(j0, rows_a, sem_a)

                    @pl.when(p < npairs - 1)
                    def _():
                        pltpu.async_copy(f_hbm.at[src_v.at[j0 + 2]], rows_a,
                                         sem_a)

                    process(j0 + 1, rows_b, sem_b)

                    @pl.when(p < npairs - 1)
                    def _():
                        pltpu.async_copy(f_hbm.at[src_v.at[j0 + 3]], rows_b,
                                         sem_b)

                    return 0

                lax.fori_loop(0, npairs, pair, 0)

            return 0

        lax.fori_loop(0, max(nph0, nph1), phase, 0)

        plsc.subcore_barrier()

        pltpu.sync_copy(acc_sh.at[pl.ds(s * rpt, rpt)],
                        agg_hbm.at[pl.ds(obase, rpt)])
        if with_deg:
            pltpu.sync_copy(dacc_sh.at[pl.ds(s * rpt, rpt)],
                            deg_hbm.at[pl.ds(obase, rpt)])

    mesh = plsc.VectorSubcoreMesh(core_axis_name="c", subcore_axis_name="s",
                                  num_cores=NC)
    return pl.kernel(body, out_type=out_type, mesh=mesh,
                     scratch_types=scratch,
                     compiler_params=pltpu.CompilerParams(
                         use_tc_tiling_on_sc=False))


def _mm_first(xp, w, b, tm):
    n_pad, d = xp.shape

    def body(x_ref, w_ref, b_ref, o_ref):
        h = jnp.dot(x_ref[...], w_ref[...], preferred_element_type=jnp.float32)
        o_ref[...] = jnp.maximum(h + b_ref[...], 0.0)

    return pl.pallas_call(
        body,
        grid=(n_pad // tm,),
        in_specs=[
            pl.BlockSpec((tm, d), lambda i: (i, 0)),
            pl.BlockSpec((d, d), lambda i: (0, 0)),
            pl.BlockSpec((1, d), lambda i: (0, 0)),
        ],
        out_specs=pl.BlockSpec((tm, d), lambda i: (i, 0)),
        out_shape=jax.ShapeDtypeStruct((n_pad, d), jnp.float32),
    )(xp, w, b.reshape(1, d))


def _mm_pool(p, dg, w, b, tm):
    _, n_pad, d = p.shape

    def body(p_ref, dg_ref, w_ref, b_ref, o_ref):
        pv = p_ref[...]
        dv = dg_ref[...]
        deg = jnp.maximum(dv[0, :, 0:1] + dv[1, :, 0:1], 1.0)
        h = (pv[0] + pv[1]) / deg
        h = jnp.dot(h, w_ref[...], preferred_element_type=jnp.float32)
        o_ref[...] = jnp.maximum(h + b_ref[...], 0.0)

    return pl.pallas_call(
        body,
        grid=(n_pad // tm,),
        in_specs=[
            pl.BlockSpec((NC, tm, d), lambda i: (0, i, 0)),
            pl.BlockSpec((NC, tm, DW), lambda i: (0, i, 0)),
            pl.BlockSpec((d, d), lambda i: (0, 0)),
            pl.BlockSpec((1, d), lambda i: (0, 0)),
        ],
        out_specs=pl.BlockSpec((tm, d), lambda i: (i, 0)),
        out_shape=jax.ShapeDtypeStruct((n_pad, d), jnp.float32),
    )(p, dg, w, b.reshape(1, d))


def _pool_final(p, dg, tm):
    _, n_pad, d = p.shape

    def body(p_ref, dg_ref, o_ref):
        pv = p_ref[...]
        dv = dg_ref[...]
        deg = jnp.maximum(dv[0, :, 0:1] + dv[1, :, 0:1], 1.0)
        o_ref[...] = (pv[0] + pv[1]) / deg

    return pl.pallas_call(
        body,
        grid=(n_pad // tm,),
        in_specs=[
            pl.BlockSpec((NC, tm, d), lambda i: (0, i, 0)),
            pl.BlockSpec((NC, tm, DW), lambda i: (0, i, 0)),
        ],
        out_specs=pl.BlockSpec((tm, d), lambda i: (i, 0)),
        out_shape=jax.ShapeDtypeStruct((n_pad, d), jnp.float32),
    )(p, dg)


def kernel(x, edge_index, W1, b1, W2, b2, W3, b3):
    n, d = x.shape
    e = edge_index.shape[1]
    n_pad = _cdiv(n, NS * CH) * NS * CH
    pc = 40
    nphases = _cdiv(e, NS * CH * pc)
    c_chunks = pc * nphases
    e_pad = c_chunks * NS * CH
    tm = n_pad // 8

    src = edge_index[0]
    dst = edge_index[1]
    srcp = jnp.concatenate(
        [src, jnp.zeros((e_pad - e,), jnp.int32)]).reshape(NS * c_chunks, CH)
    dstp = jnp.concatenate(
        [dst, jnp.full((e_pad - e,), n, jnp.int32)]).reshape(NS * c_chunks, CH)
    xp = jnp.pad(x, ((0, n_pad - n), (0, 0)))

    rpt = n_pad // NS
    zrow = jnp.zeros((rpt, d), jnp.float32)
    zdeg = jnp.zeros((rpt, DW), jnp.float32)
    ones = jnp.ones((CH, DW), jnp.float32)

    agg_deg = _make_agg(n_pad, d, 20, 14, 2, with_deg=True)
    agg = _make_agg(n_pad, d, 20, 14, 2, with_deg=False)

    f1 = _mm_first(xp, W1, b1, tm)
    p1, dg = agg_deg(f1, srcp, dstp, zrow, zdeg, ones)
    p1 = p1.reshape(NC, n_pad, d)
    dg = dg.reshape(NC, n_pad, DW)
    f2 = _mm_pool(p1, dg, W2, b2, tm)
    (p2,) = agg(f2, srcp, dstp, zrow, zdeg, ones)
    f3 = _mm_pool(p2.reshape(NC, n_pad, d), dg, W3, b3, tm)
    (p3,) = agg(f3, srcp, dstp, zrow, zdeg, ones)
    out = _pool_final(p3.reshape(NC, n_pad, d), dg, tm)
    return out[:n]


# speedup vs baseline: 1.2370x; 1.0663x over previous
import functools

import jax
import jax.numpy as jnp
from jax import lax
from jax.experimental import pallas as pl
from jax.experimental.pallas import tpu as pltpu
from jax.experimental.pallas import tpu_sc as plsc

NC = 2
NS = 16
CH = 64
DW = 16


def _cdiv(a, b):
    return (a + b - 1) // b


def _make_agg(n_pad, d, pc, nph0, nph1, with_deg):
    rpt = n_pad // NS
    npairs = pc // 2

    out_type = [jax.ShapeDtypeStruct((NC * n_pad, d), jnp.float32)]
    scratch = [
        pltpu.VMEM((pc, CH), jnp.int32),
        pltpu.VMEM((pc, CH), jnp.int32),
        pltpu.VMEM((CH, d), jnp.float32),
        pltpu.VMEM((CH, d), jnp.float32),
        pltpu.VMEM_SHARED((n_pad, d), jnp.float32),
        pltpu.SemaphoreType.DMA,
        pltpu.SemaphoreType.DMA,
    ]
    if with_deg:
        out_type.append(jax.ShapeDtypeStruct((NC * n_pad, DW), jnp.float32))
        scratch += [
            pltpu.VMEM((CH, DW), jnp.float32),
            pltpu.VMEM_SHARED((n_pad, DW), jnp.float32),
        ]

    def body(f_hbm, src_hbm, dst_hbm, zrow_hbm, zdeg_hbm, ones_hbm, *rest):
        if with_deg:
            (agg_hbm, deg_hbm, src_v, dst_v, rows_a, rows_b, acc_sh,
             sem_a, sem_b, ones_v, dacc_sh) = rest
        else:
            (agg_hbm, src_v, dst_v, rows_a, rows_b, acc_sh,
             sem_a, sem_b) = rest
            deg_hbm = ones_v = dacc_sh = None
        c = lax.axis_index("c")
        s = lax.axis_index("s")
        nph_c = jnp.where(c == 0, nph0, nph1)
        base0 = jnp.where(c == 0, s * (nph0 * pc),
                          NS * (nph0 * pc) + s * (nph1 * pc))
        obase = c * n_pad + s * rpt

        pltpu.sync_copy(zrow_hbm, acc_sh.at[pl.ds(s * rpt, rpt)])
        if with_deg:
            pltpu.sync_copy(zdeg_hbm, dacc_sh.at[pl.ds(s * rpt, rpt)])
            pltpu.sync_copy(ones_hbm, ones_v)

        plsc.subcore_barrier()

        def gather_wait(buf, sem):
            pltpu.make_async_copy(f_hbm.at[pl.ds(0, CH)], buf, sem).wait()

        def process(j, buf, sem):
            gather_wait(buf, sem)
            pltpu.sync_copy(buf, acc_sh.at[dst_v.at[j]], add=True)
            if with_deg:
                pltpu.sync_copy(ones_v, dacc_sh.at[dst_v.at[j]], add=True)

        def phase(h, _):
            @pl.when(h < nph_c)
            def _():
                base = base0 + h * pc
                pltpu.sync_copy(src_hbm.at[pl.ds(base, pc)], src_v)
                pltpu.sync_copy(dst_hbm.at[pl.ds(base, pc)], dst_v)

                pltpu.async_copy(f_hbm.at[src_v.at[0]], rows_a, sem_a)
                pltpu.async_copy(f_hbm.at[src_v.at[1]], rows_b, sem_b)

                def pair(p, _):
                    j0 = 2 * p
                    process(j0, rows_a, sem_a)

                    @pl.when(p < npairs - 1)
                    def _():
                        pltpu.async_copy(f_hbm.at[src_v.at[j0 + 2]], rows_a,
                                         sem_a)

                    process(j0 + 1, rows_b, sem_b)

                    @pl.when(p < npairs - 1)
                    def _():
                        pltpu.async_copy(f_hbm.at[src_v.at[j0 + 3]], rows_b,
                                         sem_b)

                    return 0

                lax.fori_loop(0, npairs, pair, 0)

            return 0

        lax.fori_loop(0, max(nph0, nph1), phase, 0)

        plsc.subcore_barrier()

        pltpu.sync_copy(acc_sh.at[pl.ds(s * rpt, rpt)],
                        agg_hbm.at[pl.ds(obase, rpt)])
        if with_deg:
            pltpu.sync_copy(dacc_sh.at[pl.ds(s * rpt, rpt)],
                            deg_hbm.at[pl.ds(obase, rpt)])

    mesh = plsc.VectorSubcoreMesh(core_axis_name="c", subcore_axis_name="s",
                                  num_cores=NC)
    return pl.kernel(body, out_type=out_type, mesh=mesh,
                     scratch_types=scratch,
                     compiler_params=pltpu.CompilerParams(
                         use_tc_tiling_on_sc=False))


def _mm_first(xp, w, b, tm):
    n_pad, d = xp.shape

    def body(x_ref, w_ref, b_ref, o_ref):
        h = jnp.dot(x_ref[...], w_ref[...], preferred_element_type=jnp.float32)
        o_ref[...] = jnp.maximum(h + b_ref[...], 0.0)

    return pl.pallas_call(
        body,
        grid=(n_pad // tm,),
        in_specs=[
            pl.BlockSpec((tm, d), lambda i: (i, 0)),
            pl.BlockSpec((d, d), lambda i: (0, 0)),
            pl.BlockSpec((1, d), lambda i: (0, 0)),
        ],
        out_specs=pl.BlockSpec((tm, d), lambda i: (i, 0)),
        out_shape=jax.ShapeDtypeStruct((n_pad, d), jnp.float32),
    )(xp, w, b.reshape(1, d))


def _mm_pool(p, dg, w, b, tm):
    _, n_pad, d = p.shape

    def body(p_ref, dg_ref, w_ref, b_ref, o_ref):
        pv = p_ref[...]
        dv = dg_ref[...]
        deg = jnp.maximum(dv[0, :, 0:1] + dv[1, :, 0:1], 1.0)
        h = (pv[0] + pv[1]) / deg
        h = jnp.dot(h, w_ref[...], preferred_element_type=jnp.float32)
        o_ref[...] = jnp.maximum(h + b_ref[...], 0.0)

    return pl.pallas_call(
        body,
        grid=(n_pad // tm,),
        in_specs=[
            pl.BlockSpec((NC, tm, d), lambda i: (0, i, 0)),
            pl.BlockSpec((NC, tm, DW), lambda i: (0, i, 0)),
            pl.BlockSpec((d, d), lambda i: (0, 0)),
            pl.BlockSpec((1, d), lambda i: (0, 0)),
        ],
        out_specs=pl.BlockSpec((tm, d), lambda i: (i, 0)),
        out_shape=jax.ShapeDtypeStruct((n_pad, d), jnp.float32),
    )(p, dg, w, b.reshape(1, d))


def _pool_final(p, dg, tm):
    _, n_pad, d = p.shape

    def body(p_ref, dg_ref, o_ref):
        pv = p_ref[...]
        dv = dg_ref[...]
        deg = jnp.maximum(dv[0, :, 0:1] + dv[1, :, 0:1], 1.0)
        o_ref[...] = (pv[0] + pv[1]) / deg

    return pl.pallas_call(
        body,
        grid=(n_pad // tm,),
        in_specs=[
            pl.BlockSpec((NC, tm, d), lambda i: (0, i, 0)),
            pl.BlockSpec((NC, tm, DW), lambda i: (0, i, 0)),
        ],
        out_specs=pl.BlockSpec((tm, d), lambda i: (i, 0)),
        out_shape=jax.ShapeDtypeStruct((n_pad, d), jnp.float32),
    )(p, dg)


def kernel(x, edge_index, W1, b1, W2, b2, W3, b3):
    n, d = x.shape
    e = edge_index.shape[1]
    n_pad = _cdiv(n, NS * CH) * NS * CH
    pc = 40
    nphases = _cdiv(e, NS * CH * pc)
    c_chunks = pc * nphases
    e_pad = c_chunks * NS * CH
    tm = n_pad // 8

    src = edge_index[0]
    dst = edge_index[1]
    srcp = jnp.concatenate(
        [src, jnp.zeros((e_pad - e,), jnp.int32)]).reshape(NS * c_chunks, CH)
    dstp = jnp.concatenate(
        [dst, jnp.full((e_pad - e,), n, jnp.int32)]).reshape(NS * c_chunks, CH)
    xp = jnp.pad(x, ((0, n_pad - n), (0, 0)))

    rpt = n_pad // NS
    zrow = jnp.zeros((rpt, d), jnp.float32)
    zdeg = jnp.zeros((rpt, DW), jnp.float32)
    ones = jnp.ones((CH, DW), jnp.float32)

    agg_deg = _make_agg(n_pad, d, 20, 15, 1, with_deg=True)
    agg = _make_agg(n_pad, d, 20, 15, 1, with_deg=False)

    f1 = _mm_first(xp, W1, b1, tm)
    p1, dg = agg_deg(f1, srcp, dstp, zrow, zdeg, ones)
    p1 = p1.reshape(NC, n_pad, d)
    dg = dg.reshape(NC, n_pad, DW)
    f2 = _mm_pool(p1, dg, W2, b2, tm)
    (p2,) = agg(f2, srcp, dstp, zrow, zdeg, ones)
    f3 = _mm_pool(p2.reshape(NC, n_pad, d), dg, W3, b3, tm)
    (p3,) = agg(f3, srcp, dstp, zrow, zdeg, ones)
    out = _pool_final(p3.reshape(NC, n_pad, d), dg, tm)
    return out[:n]
